# baseline (speedup 1.0000x reference)
import functools

import jax
import jax.numpy as jnp
from jax import lax
from jax.experimental import pallas as pl
from jax.experimental.pallas import tpu as pltpu
from jax.experimental.pallas import tpu_sc as plsc

_B, _T, _D = 2, 8192, 768
_H, _DH = 12, 64
_HID = 2048
_C = 1024
_EPS = 1e-6
_ROWS = _B * _C
_NW = 32
_RPW = _ROWS // _NW
_HB = 2
_HBK = _HID // _HB


def _router_body(x_ref, w_ref, s_ref):
    xb = x_ref[...]
    s = jnp.sum(xb * w_ref[...], axis=-1)
    s_ref[...] = s.reshape(1, 8, 128)


def _router(x_flat, wr):
    return pl.pallas_call(
        _router_body,
        grid=(16,),
        in_specs=[
            pl.BlockSpec((1024, _D), lambda t: (t, 0)),
            pl.BlockSpec((1, _D), lambda t: (0, 0)),
        ],
        out_specs=pl.BlockSpec((1, 8, 128), lambda t: (t, 0, 0)),
        out_shape=jax.ShapeDtypeStruct((16, 8, 128), jnp.float32),
    )(x_flat, wr)


def _cumsum_flat(m):
    r = m.reshape(128, 128)
    ii = lax.broadcasted_iota(jnp.int32, (128, 128), 0)
    jj = lax.broadcasted_iota(jnp.int32, (128, 128), 1)
    upper = (ii <= jj).astype(jnp.float32)
    rc = jnp.dot(r, upper, preferred_element_type=jnp.float32)
    same_b = (ii // 64) == (jj // 64)
    strict = ((jj < ii) & same_b).astype(jnp.float32)
    off = jnp.dot(strict, rc[:, 127:128], preferred_element_type=jnp.float32)
    return (rc + off).reshape(2, 64, 128)


def _topk_body(s_ref, idx_ref):
    minint = jnp.int32(-2147483648)
    cf = jnp.int32(_C)
    s = s_ref[...]
    u = lax.bitcast_convert_type(s, jnp.int32)
    skey = jnp.where(u < 0, jnp.invert(u) ^ minint, u)
    cnt_pos = jnp.sum((skey >= 0).astype(jnp.int32), axis=(1, 2), keepdims=True)
    base0 = jnp.where(cnt_pos >= cf, jnp.int32(0), minint)

    def bit_body(i, base):
        cand = base | (jnp.int32(1) << (jnp.int32(30) - i))
        cnt = jnp.sum((skey >= cand).astype(jnp.int32), axis=(1, 2), keepdims=True)
        return jnp.where(cnt >= cf, cand, base)

    t = lax.fori_loop(0, 31, bit_body, base0)
    gt = skey > t
    n_gt = jnp.sum(gt.astype(jnp.int32), axis=(1, 2), keepdims=True)
    eq = skey == t
    eq_cum = _cumsum_flat(eq.astype(jnp.float32))
    take = eq & (eq_cum <= (cf - n_gt).astype(jnp.float32))
    sel = (gt | take).astype(jnp.float32)
    S = _cumsum_flat(sel)
    crange = lax.broadcasted_iota(jnp.int32, (2, _C, 1), 1).astype(jnp.float32)
    totals = jnp.sum(sel, axis=2)
    i64 = lax.broadcasted_iota(jnp.int32, (64, 64), 0)
    j64 = lax.broadcasted_iota(jnp.int32, (64, 64), 1)
    u64 = (i64 <= j64).astype(jnp.float32)
    send = jnp.dot(totals, u64,
                   preferred_element_type=jnp.float32).reshape(2, 1, 64)
    row_of = jnp.sum((send <= crange).astype(jnp.float32), axis=2,
                     keepdims=True)
    rr = lax.broadcasted_iota(jnp.int32, (2, _C, 64), 2).astype(jnp.float32)
    onehot = (row_of == rr).astype(jnp.float32)
    s_hi = jnp.floor(S * (1.0 / 256.0))
    s_lo = S - 256.0 * s_hi
    rows = jnp.stack(
        [256.0 * jnp.dot(onehot[b], s_hi[b], preferred_element_type=jnp.float32)
         + jnp.dot(onehot[b], s_lo[b], preferred_element_type=jnp.float32)
         for b in range(2)], axis=0)
    within = jnp.sum((rows <= crange).astype(jnp.float32), axis=2)
    row_of2 = jnp.sum((send <= crange).astype(jnp.float32), axis=2)
    acc = 128.0 * row_of2 + within
    boff = lax.broadcasted_iota(jnp.int32, (2, _C), 0) * jnp.int32(_T)
    idx_ref[...] = acc.astype(jnp.int32) + boff


def _topk(scores):
    return pl.pallas_call(
        _topk_body,
        out_shape=jax.ShapeDtypeStruct((2, _C), jnp.int32),
    )(scores)


@functools.lru_cache(maxsize=None)
def _build_sc_kernels():
    mesh = plsc.VectorSubcoreMesh(core_axis_name="c", subcore_axis_name="s")
    scratch = [
        pltpu.VMEM((_RPW,), jnp.int32),
        pltpu.VMEM((_RPW, _D), jnp.float32),
        pltpu.SemaphoreType.DMA,
    ]

    @functools.partial(
        pl.kernel,
        out_type=jax.ShapeDtypeStruct((_ROWS, _D), jnp.float32),
        mesh=mesh,
        scratch_types=scratch,
    )
    def gather(x_hbm, idx_hbm, out_hbm, idx_v, rows_v, sem):
        wid = lax.axis_index("s") * 2 + lax.axis_index("c")
        base = wid * _RPW
        pltpu.sync_copy(idx_hbm.at[pl.ds(base, _RPW)], idx_v)
        pltpu.async_copy(x_hbm.at[idx_v], rows_v, sem).wait()
        pltpu.sync_copy(rows_v, out_hbm.at[pl.ds(base, _RPW)])

    @functools.partial(pl.kernel, out_type=(), mesh=mesh, scratch_types=scratch)
    def scatter(out_ref, rows_hbm, idx_hbm, idx_v, rows_v, sem):
        wid = lax.axis_index("s") * 2 + lax.axis_index("c")
        base = wid * _RPW
        pltpu.sync_copy(idx_hbm.at[pl.ds(base, _RPW)], idx_v)
        pltpu.sync_copy(rows_hbm.at[pl.ds(base, _RPW)], rows_v)
        pltpu.async_copy(rows_v, out_ref.at[idx_v], sem).wait()

    crows = (_B * _T) // _NW

    @functools.partial(
        pl.kernel,
        out_type=jax.ShapeDtypeStruct((_B * _T, _D), jnp.float32),
        mesh=mesh,
        scratch_types=[pltpu.SemaphoreType.DMA],
    )
    def copy(x_hbm, out_hbm, sem):
        wid = lax.axis_index("s") * 2 + lax.axis_index("c")
        base = wid * crows
        pltpu.async_copy(x_hbm.at[pl.ds(base, crows)],
                         out_hbm.at[pl.ds(base, crows)], sem).wait()

    return gather, scatter, copy


def _sc_gather(x_flat, idx_flat):
    return _build_sc_kernels()[0](x_flat, idx_flat)


def _sc_scatter(out_ref, rows, idx_flat):
    return _build_sc_kernels()[1](out_ref, rows, idx_flat)


def _sc_copy(x_flat):
    return _build_sc_kernels()[2](x_flat)


def _proj_body(sel_ref, g1_ref, wq_ref, wk_ref, wv_ref, q_ref, k_ref, v_ref):
    xb = sel_ref[0]
    ms = jnp.mean(xb * xb, axis=-1, keepdims=True)
    xn = (xb * lax.rsqrt(ms + _EPS) * g1_ref[...]).astype(jnp.bfloat16)
    q = (jnp.dot(xn, wq_ref[...], preferred_element_type=jnp.float32)
         * 0.125).astype(jnp.bfloat16)
    k = jnp.dot(xn, wk_ref[...],
                preferred_element_type=jnp.float32).astype(jnp.bfloat16)
    v = jnp.dot(xn, wv_ref[...],
                preferred_element_type=jnp.float32).astype(jnp.bfloat16)
    for h in range(_H):
        q_ref[0, h] = q[:, h * _DH:(h + 1) * _DH]
        k_ref[0, h] = k[:, h * _DH:(h + 1) * _DH]
        v_ref[0, h] = v[:, h * _DH:(h + 1) * _DH]


def _proj(sel3, g1r, wq16, wk16, wv16):
    hspec = jax.ShapeDtypeStruct((_B, _H, _C, _DH), jnp.bfloat16)
    return pl.pallas_call(
        _proj_body,
        grid=(_B,),
        in_specs=[
            pl.BlockSpec((1, _C, _D), lambda b: (b, 0, 0)),
            pl.BlockSpec((1, _D), lambda b: (0, 0)),
            pl.BlockSpec((_D, _D), lambda b: (0, 0)),
            pl.BlockSpec((_D, _D), lambda b: (0, 0)),
            pl.BlockSpec((_D, _D), lambda b: (0, 0)),
        ],
        out_specs=[pl.BlockSpec((1, _H, _C, _DH), lambda b: (b, 0, 0, 0))] * 3,
        out_shape=[hspec, hspec, hspec],
    )(sel3, g1r, wq16, wk16, wv16)


_RB = 256
_NRB = _C // _RB


def _attn_body(q_ref, k_ref, v_ref, o_ref, bias_ref):
    b = pl.program_id(0)
    h = pl.program_id(1)

    @pl.when((b == 0) & (h == 0))
    def _():
        ii = lax.broadcasted_iota(jnp.int32, (_RB, _RB), 0)
        jj = lax.broadcasted_iota(jnp.int32, (_RB, _RB), 1)
        bias_ref[...] = jnp.where(ii >= jj, jnp.float32(0), jnp.float32(-1e9))

    q = q_ref[0, 0]
    k = k_ref[0, 0]
    v = v_ref[0, 0]
    bias = bias_ref[...]
    for rb in range(_NRB):
        qb = q[rb * _RB:(rb + 1) * _RB]
        kd = k[rb * _RB:(rb + 1) * _RB]
        vd = v[rb * _RB:(rb + 1) * _RB]
        attd = lax.dot_general(qb, kd, (((1,), (1,)), ((), ())),
                               preferred_element_type=jnp.float32) + bias
        if rb == 0:
            m = jnp.max(attd, axis=-1, keepdims=True)
            p = jnp.exp(attd - m)
            num = jnp.dot(p.astype(jnp.bfloat16), vd,
                          preferred_element_type=jnp.float32)
            den = jnp.sum(p, axis=-1, keepdims=True)
        else:
            kf = k[:rb * _RB]
            vf = v[:rb * _RB]
            attf = lax.dot_general(qb, kf, (((1,), (1,)), ((), ())),
                                   preferred_element_type=jnp.float32)
            m = jnp.maximum(jnp.max(attf, axis=-1, keepdims=True),
                            jnp.max(attd, axis=-1, keepdims=True))
            pf = jnp.exp(attf - m)
            pd = jnp.exp(attd - m)
            num = (jnp.dot(pf.astype(jnp.bfloat16), vf,
                           preferred_element_type=jnp.float32)
                   + jnp.dot(pd.astype(jnp.bfloat16), vd,
                             preferred_element_type=jnp.float32))
            den = (jnp.sum(pf, axis=-1, keepdims=True)
                   + jnp.sum(pd, axis=-1, keepdims=True))
        o_ref[0, 0, rb * _RB:(rb + 1) * _RB] = (num / den).astype(jnp.bfloat16)


def _attn(q4, k4, v4):
    hspec = pl.BlockSpec((1, 1, _C, _DH), lambda b, h: (b, h, 0, 0))
    return pl.pallas_call(
        _attn_body,
        grid=(_B, _H),
        in_specs=[hspec, hspec, hspec],
        out_specs=pl.BlockSpec((1, 1, _C, _DH), lambda b, h: (b, h, 0, 0)),
        out_shape=jax.ShapeDtypeStruct((_B, _H, _C, _DH), jnp.bfloat16),
        scratch_shapes=[pltpu.VMEM((_RB, _RB), jnp.float32)],
    )(q4, k4, v4)


def _mlp_body(sel_ref, o_ref, wo_ref, g2_ref, w1_ref, w3_ref, w2_ref,
              out_ref, res_ref, y_ref, acc_ref):
    hb = pl.program_id(1)

    @pl.when(hb == 0)
    def _():
        res = sel_ref[0]
        for h in range(_H):
            res = res + jnp.dot(o_ref[0, h], wo_ref[h],
                                preferred_element_type=jnp.float32)
        res_ref[...] = res
        ms = jnp.mean(res * res, axis=-1, keepdims=True)
        y = res * lax.rsqrt(ms + _EPS) * g2_ref[...]
        y_ref[...] = y.astype(jnp.bfloat16)
        acc_ref[...] = jnp.zeros_like(acc_ref)

    y = y_ref[...]
    a = jnp.dot(y, w1_ref[...], preferred_element_type=jnp.float32)
    g = jnp.dot(y, w3_ref[...], preferred_element_type=jnp.float32)
    sa = (a / (1.0 + jnp.exp(-a)) * g).astype(jnp.bfloat16)
    acc_ref[...] += jnp.dot(sa, w2_ref[...], preferred_element_type=jnp.float32)

    @pl.when(hb == _HB - 1)
    def _():
        out_ref[0] = res_ref[...] + acc_ref[...]


def _mlp(sel3, o4, wo_r, g2r, w1, w3, w2):
    return pl.pallas_call(
        _mlp_body,
        grid=(_B, _HB),
        in_specs=[
            pl.BlockSpec((1, _C, _D), lambda b, hb: (b, 0, 0)),
            pl.BlockSpec((1, _H, _C, _DH), lambda b, hb: (b, 0, 0, 0)),
            pl.BlockSpec((_H, _DH, _D), lambda b, hb: (0, 0, 0)),
            pl.BlockSpec((1, _D), lambda b, hb: (0, 0)),
            pl.BlockSpec((_D, _HBK), lambda b, hb: (0, hb)),
            pl.BlockSpec((_D, _HBK), lambda b, hb: (0, hb)),
            pl.BlockSpec((_HBK, _D), lambda b, hb: (hb, 0)),
        ],
        out_specs=pl.BlockSpec((1, _C, _D), lambda b, hb: (b, 0, 0)),
        out_shape=jax.ShapeDtypeStruct((_B, _C, _D), jnp.float32),
        scratch_shapes=[
            pltpu.VMEM((_C, _D), jnp.float32),
            pltpu.VMEM((_C, _D), jnp.bfloat16),
            pltpu.VMEM((_C, _D), jnp.float32),
        ],
    )(sel3, o4, wo_r, g2r, w1, w3, w2)


def kernel(x, w_router, b_router, g1, g2, wq, wk, wv, wo, w1, w3, w2):
    x_flat = x.reshape(_B * _T, _D)
    scores = _router(x_flat, w_router.reshape(1, _D))
    idx2 = _topk(scores.reshape(2, 64, 128))
    idx_flat = idx2.reshape(_ROWS)
    sel = _sc_gather(x_flat, idx_flat)
    out_flat = _sc_copy(x_flat)
    sel3 = sel.reshape(_B, _C, _D)
    bf = jnp.bfloat16
    q4, k4, v4 = _proj(sel3, g1.reshape(1, _D),
                       wq.astype(bf), wk.astype(bf), wv.astype(bf))
    o4 = _attn(q4, k4, v4)
    upd = _mlp(sel3, o4, wo.reshape(_H, _DH, _D).astype(bf), g2.reshape(1, _D),
               w1.astype(bf), w3.astype(bf), w2.astype(bf))
    out_ref = jax.new_ref(out_flat)
    _sc_scatter(out_ref, upd.reshape(_ROWS, _D), idx_flat)
    return jax.freeze(out_ref).reshape(_B, _T, _D)


# speedup vs baseline: 8.5573x; 8.5573x over previous
import functools

import jax
import jax.numpy as jnp
from jax import lax
from jax.experimental import pallas as pl
from jax.experimental.pallas import tpu as pltpu
from jax.experimental.pallas import tpu_sc as plsc

_B, _T, _D = 2, 8192, 768
_H, _DH = 12, 64
_HID = 2048
_C = 1024
_EPS = 1e-6
_ROWS = _B * _C
_NW = 32
_RPW = _ROWS // _NW
_HB = 2
_HBK = _HID // _HB


def _router_copy_body(x_ref, w_ref, out_ref, s_ref):
    xb = x_ref[...]
    out_ref[...] = xb
    s = jnp.sum(xb * w_ref[...], axis=-1)
    s_ref[...] = s.reshape(1, 8, 128)


def _router_copy(x_flat, wr):
    return pl.pallas_call(
        _router_copy_body,
        grid=(16,),
        in_specs=[
            pl.BlockSpec((1024, _D), lambda t: (t, 0)),
            pl.BlockSpec((1, _D), lambda t: (0, 0)),
        ],
        out_specs=[
            pl.BlockSpec((1024, _D), lambda t: (t, 0)),
            pl.BlockSpec((1, 8, 128), lambda t: (t, 0, 0)),
        ],
        out_shape=[
            jax.ShapeDtypeStruct((_B * _T, _D), jnp.float32),
            jax.ShapeDtypeStruct((16, 8, 128), jnp.float32),
        ],
    )(x_flat, wr)


def _cumsum_flat(m):
    r = m.reshape(128, 128)
    ii = lax.broadcasted_iota(jnp.int32, (128, 128), 0)
    jj = lax.broadcasted_iota(jnp.int32, (128, 128), 1)
    upper = (ii <= jj).astype(jnp.float32)
    rc = jnp.dot(r, upper, preferred_element_type=jnp.float32)
    same_b = (ii // 64) == (jj // 64)
    strict = ((jj < ii) & same_b).astype(jnp.float32)
    off = jnp.dot(strict, rc[:, 127:128], preferred_element_type=jnp.float32)
    return (rc + off).reshape(2, 64, 128)


def _topk_body(s_ref, idx_ref):
    minint = jnp.int32(-2147483648)
    cf = jnp.int32(_C)
    s = s_ref[...]
    u = lax.bitcast_convert_type(s, jnp.int32)
    skey = jnp.where(u < 0, jnp.invert(u) ^ minint, u)
    cnt_pos = jnp.sum((skey >= 0).astype(jnp.int32), axis=(1, 2), keepdims=True)
    base0 = jnp.where(cnt_pos >= cf, jnp.int32(0), minint)

    def bit_body(i, base):
        cand = base | (jnp.int32(1) << (jnp.int32(30) - i))
        cnt = jnp.sum((skey >= cand).astype(jnp.int32), axis=(1, 2), keepdims=True)
        return jnp.where(cnt >= cf, cand, base)

    t = lax.fori_loop(0, 31, bit_body, base0)
    gt = skey > t
    n_gt = jnp.sum(gt.astype(jnp.int32), axis=(1, 2), keepdims=True)
    eq = skey == t
    eq_cum = _cumsum_flat(eq.astype(jnp.float32))
    take = eq & (eq_cum <= (cf - n_gt).astype(jnp.float32))
    sel = (gt | take).astype(jnp.float32)
    S = _cumsum_flat(sel)
    crange = lax.broadcasted_iota(jnp.int32, (2, _C, 1), 1).astype(jnp.float32)
    totals = jnp.sum(sel, axis=2)
    i64 = lax.broadcasted_iota(jnp.int32, (64, 64), 0)
    j64 = lax.broadcasted_iota(jnp.int32, (64, 64), 1)
    u64 = (i64 <= j64).astype(jnp.float32)
    send = jnp.dot(totals, u64,
                   preferred_element_type=jnp.float32).reshape(2, 1, 64)
    row_of = jnp.sum((send <= crange).astype(jnp.float32), axis=2,
                     keepdims=True)
    rr = lax.broadcasted_iota(jnp.int32, (2, _C, 64), 2).astype(jnp.float32)
    onehot = (row_of == rr).astype(jnp.float32)
    s_hi = jnp.floor(S * (1.0 / 256.0))
    s_lo = S - 256.0 * s_hi
    rows = jnp.stack(
        [256.0 * jnp.dot(onehot[b], s_hi[b], preferred_element_type=jnp.float32)
         + jnp.dot(onehot[b], s_lo[b], preferred_element_type=jnp.float32)
         for b in range(2)], axis=0)
    within = jnp.sum((rows <= crange).astype(jnp.float32), axis=2)
    row_of2 = jnp.sum((send <= crange).astype(jnp.float32), axis=2)
    acc = 128.0 * row_of2 + within
    boff = lax.broadcasted_iota(jnp.int32, (2, _C), 0) * jnp.int32(_T)
    idx_ref[...] = acc.astype(jnp.int32) + boff


def _topk(scores):
    return pl.pallas_call(
        _topk_body,
        out_shape=jax.ShapeDtypeStruct((2, _C), jnp.int32),
    )(scores)


@functools.lru_cache(maxsize=None)
def _build_sc_kernels():
    mesh = plsc.VectorSubcoreMesh(core_axis_name="c", subcore_axis_name="s")
    scratch = [
        pltpu.VMEM((_RPW,), jnp.int32),
        pltpu.VMEM((_RPW, _D), jnp.float32),
        pltpu.SemaphoreType.DMA,
    ]

    @functools.partial(
        pl.kernel,
        out_type=jax.ShapeDtypeStruct((_ROWS, _D), jnp.float32),
        mesh=mesh,
        scratch_types=scratch,
    )
    def gather(x_hbm, idx_hbm, out_hbm, idx_v, rows_v, sem):
        wid = lax.axis_index("s") * 2 + lax.axis_index("c")
        base = wid * _RPW
        pltpu.sync_copy(idx_hbm.at[pl.ds(base, _RPW)], idx_v)
        pltpu.async_copy(x_hbm.at[idx_v], rows_v, sem).wait()
        pltpu.sync_copy(rows_v, out_hbm.at[pl.ds(base, _RPW)])

    @functools.partial(pl.kernel, out_type=(), mesh=mesh, scratch_types=scratch)
    def scatter(out_ref, rows_hbm, idx_hbm, idx_v, rows_v, sem):
        wid = lax.axis_index("s") * 2 + lax.axis_index("c")
        base = wid * _RPW
        pltpu.sync_copy(idx_hbm.at[pl.ds(base, _RPW)], idx_v)
        pltpu.sync_copy(rows_hbm.at[pl.ds(base, _RPW)], rows_v)
        pltpu.async_copy(rows_v, out_ref.at[idx_v], sem).wait()

    return gather, scatter


def _sc_gather(x_flat, idx_flat):
    return _build_sc_kernels()[0](x_flat, idx_flat)


def _sc_scatter(out_ref, rows, idx_flat):
    return _build_sc_kernels()[1](out_ref, rows, idx_flat)


def _proj_body(sel_ref, g1_ref, wq_ref, wk_ref, wv_ref, q_ref, k_ref, v_ref):
    xb = sel_ref[0]
    ms = jnp.mean(xb * xb, axis=-1, keepdims=True)
    xn = (xb * lax.rsqrt(ms + _EPS) * g1_ref[...]).astype(jnp.bfloat16)
    q = (jnp.dot(xn, wq_ref[...], preferred_element_type=jnp.float32)
         * 0.125).astype(jnp.bfloat16)
    k = jnp.dot(xn, wk_ref[...],
                preferred_element_type=jnp.float32).astype(jnp.bfloat16)
    v = jnp.dot(xn, wv_ref[...],
                preferred_element_type=jnp.float32).astype(jnp.bfloat16)
    for h in range(_H):
        q_ref[0, h] = q[:, h * _DH:(h + 1) * _DH]
        k_ref[0, h] = k[:, h * _DH:(h + 1) * _DH]
        v_ref[0, h] = v[:, h * _DH:(h + 1) * _DH]


def _proj(sel3, g1r, wq16, wk16, wv16):
    hspec = jax.ShapeDtypeStruct((_B, _H, _C, _DH), jnp.bfloat16)
    return pl.pallas_call(
        _proj_body,
        grid=(_B,),
        in_specs=[
            pl.BlockSpec((1, _C, _D), lambda b: (b, 0, 0)),
            pl.BlockSpec((1, _D), lambda b: (0, 0)),
            pl.BlockSpec((_D, _D), lambda b: (0, 0)),
            pl.BlockSpec((_D, _D), lambda b: (0, 0)),
            pl.BlockSpec((_D, _D), lambda b: (0, 0)),
        ],
        out_specs=[pl.BlockSpec((1, _H, _C, _DH), lambda b: (b, 0, 0, 0))] * 3,
        out_shape=[hspec, hspec, hspec],
    )(sel3, g1r, wq16, wk16, wv16)


_RB = 256
_NRB = _C // _RB


def _attn_body(q_ref, k_ref, v_ref, o_ref, bias_ref):
    b = pl.program_id(0)
    h = pl.program_id(1)

    @pl.when((b == 0) & (h == 0))
    def _():
        ii = lax.broadcasted_iota(jnp.int32, (_RB, _RB), 0)
        jj = lax.broadcasted_iota(jnp.int32, (_RB, _RB), 1)
        bias_ref[...] = jnp.where(ii >= jj, jnp.float32(0), jnp.float32(-1e9))

    q = q_ref[0, 0]
    k = k_ref[0, 0]
    v = v_ref[0, 0]
    bias = bias_ref[...]
    for rb in range(_NRB):
        qb = q[rb * _RB:(rb + 1) * _RB]
        kd = k[rb * _RB:(rb + 1) * _RB]
        vd = v[rb * _RB:(rb + 1) * _RB]
        attd = lax.dot_general(qb, kd, (((1,), (1,)), ((), ())),
                               preferred_element_type=jnp.float32) + bias
        if rb == 0:
            m = jnp.max(attd, axis=-1, keepdims=True)
            p = jnp.exp(attd - m)
            num = jnp.dot(p.astype(jnp.bfloat16), vd,
                          preferred_element_type=jnp.float32)
            den = jnp.sum(p, axis=-1, keepdims=True)
        else:
            kf = k[:rb * _RB]
            vf = v[:rb * _RB]
            attf = lax.dot_general(qb, kf, (((1,), (1,)), ((), ())),
                                   preferred_element_type=jnp.float32)
            m = jnp.maximum(jnp.max(attf, axis=-1, keepdims=True),
                            jnp.max(attd, axis=-1, keepdims=True))
            pf = jnp.exp(attf - m)
            pd = jnp.exp(attd - m)
            num = (jnp.dot(pf.astype(jnp.bfloat16), vf,
                           preferred_element_type=jnp.float32)
                   + jnp.dot(pd.astype(jnp.bfloat16), vd,
                             preferred_element_type=jnp.float32))
            den = (jnp.sum(pf, axis=-1, keepdims=True)
                   + jnp.sum(pd, axis=-1, keepdims=True))
        o_ref[0, 0, rb * _RB:(rb + 1) * _RB] = (num / den).astype(jnp.bfloat16)


def _attn(q4, k4, v4):
    hspec = pl.BlockSpec((1, 1, _C, _DH), lambda b, h: (b, h, 0, 0))
    return pl.pallas_call(
        _attn_body,
        grid=(_B, _H),
        in_specs=[hspec, hspec, hspec],
        out_specs=pl.BlockSpec((1, 1, _C, _DH), lambda b, h: (b, h, 0, 0)),
        out_shape=jax.ShapeDtypeStruct((_B, _H, _C, _DH), jnp.bfloat16),
        scratch_shapes=[pltpu.VMEM((_RB, _RB), jnp.float32)],
    )(q4, k4, v4)


def _mlp_body(sel_ref, o_ref, wo_ref, g2_ref, w1_ref, w3_ref, w2_ref,
              out_ref, res_ref, y_ref, acc_ref):
    hb = pl.program_id(1)

    @pl.when(hb == 0)
    def _():
        res = sel_ref[0]
        for h in range(_H):
            res = res + jnp.dot(o_ref[0, h], wo_ref[h],
                                preferred_element_type=jnp.float32)
        res_ref[...] = res
        ms = jnp.mean(res * res, axis=-1, keepdims=True)
        y = res * lax.rsqrt(ms + _EPS) * g2_ref[...]
        y_ref[...] = y.astype(jnp.bfloat16)
        acc_ref[...] = jnp.zeros_like(acc_ref)

    y = y_ref[...]
    a = jnp.dot(y, w1_ref[...], preferred_element_type=jnp.float32)
    g = jnp.dot(y, w3_ref[...], preferred_element_type=jnp.float32)
    sa = (a / (1.0 + jnp.exp(-a)) * g).astype(jnp.bfloat16)
    acc_ref[...] += jnp.dot(sa, w2_ref[...], preferred_element_type=jnp.float32)

    @pl.when(hb == _HB - 1)
    def _():
        out_ref[0] = res_ref[...] + acc_ref[...]


def _mlp(sel3, o4, wo_r, g2r, w1, w3, w2):
    return pl.pallas_call(
        _mlp_body,
        grid=(_B, _HB),
        in_specs=[
            pl.BlockSpec((1, _C, _D), lambda b, hb: (b, 0, 0)),
            pl.BlockSpec((1, _H, _C, _DH), lambda b, hb: (b, 0, 0, 0)),
            pl.BlockSpec((_H, _DH, _D), lambda b, hb: (0, 0, 0)),
            pl.BlockSpec((1, _D), lambda b, hb: (0, 0)),
            pl.BlockSpec((_D, _HBK), lambda b, hb: (0, hb)),
            pl.BlockSpec((_D, _HBK), lambda b, hb: (0, hb)),
            pl.BlockSpec((_HBK, _D), lambda b, hb: (hb, 0)),
        ],
        out_specs=pl.BlockSpec((1, _C, _D), lambda b, hb: (b, 0, 0)),
        out_shape=jax.ShapeDtypeStruct((_B, _C, _D), jnp.float32),
        scratch_shapes=[
            pltpu.VMEM((_C, _D), jnp.float32),
            pltpu.VMEM((_C, _D), jnp.bfloat16),
            pltpu.VMEM((_C, _D), jnp.float32),
        ],
    )(sel3, o4, wo_r, g2r, w1, w3, w2)


def kernel(x, w_router, b_router, g1, g2, wq, wk, wv, wo, w1, w3, w2):
    x_flat = x.reshape(_B * _T, _D)
    out_flat, scores = _router_copy(x_flat, w_router.reshape(1, _D))
    idx2 = _topk(scores.reshape(2, 64, 128))
    idx_flat = idx2.reshape(_ROWS)
    sel = _sc_gather(x_flat, idx_flat)
    sel3 = sel.reshape(_B, _C, _D)
    bf = jnp.bfloat16
    q4, k4, v4 = _proj(sel3, g1.reshape(1, _D),
                       wq.astype(bf), wk.astype(bf), wv.astype(bf))
    o4 = _attn(q4, k4, v4)
    upd = _mlp(sel3, o4, wo.reshape(_H, _DH, _D).astype(bf), g2.reshape(1, _D),
               w1.astype(bf), w3.astype(bf), w2.astype(bf))
    out_ref = jax.new_ref(out_flat)
    _sc_scatter(out_ref, upd.reshape(_ROWS, _D), idx_flat)
    return jax.freeze(out_ref).reshape(_B, _T, _D)
